# baseline (speedup 1.0000x reference)
import functools

import jax
import jax.numpy as jnp
from jax import lax
from jax.experimental import pallas as pl
from jax.experimental.pallas import tpu as pltpu
from jax.experimental.pallas import tpu_sc as plsc

NC, NS = 2, 16
NW = NC * NS

D = 300
DP = 320
BCH = 10
LSEG = 50
QSEG = 4
QIDX = QSEG * LSEG
NBUF = 2
FLUSH = 8


def _sc_pool(idx_flat, emb16):
    nseg = idx_flat.shape[0] // LSEG
    seg_w = nseg // NW
    q_w = seg_w // QSEG
    nblk = q_w // NBUF

    mesh = plsc.VectorSubcoreMesh(core_axis_name="c", subcore_axis_name="s")

    @functools.partial(
        pl.kernel,
        mesh=mesh,
        compiler_params=pltpu.CompilerParams(use_tc_tiling_on_sc=False),
        out_type=jax.ShapeDtypeStruct((nseg, DP), jnp.bfloat16),
        scratch_types=[
            pltpu.VMEM((seg_w * LSEG,), jnp.int32),
            pltpu.VMEM((QIDX, DP), jnp.bfloat16),
            pltpu.VMEM((QIDX, DP), jnp.bfloat16),
            pltpu.VMEM((FLUSH, DP), jnp.bfloat16),
            pltpu.SemaphoreType.DMA,
            pltpu.SemaphoreType.DMA,
        ],
    )
    def pool(idx_hbm, emb_hbm, out_hbm, idx_v, g0, g1, res_v, s0, s1):
        gbufs = (g0, g1)
        sems = (s0, s1)
        wid = lax.axis_index("s") * NC + lax.axis_index("c")
        seg_base = wid * seg_w
        pltpu.sync_copy(idx_hbm.at[pl.ds(seg_base * LSEG, seg_w * LSEG)],
                        idx_v)

        def fire(lq, j):
            pltpu.async_copy(
                emb_hbm.at[idx_v.at[pl.ds(lq * QIDX, QIDX)]], gbufs[j],
                sems[j])

        for j in range(NBUF):
            fire(j, j)

        def reduce_seg(gk, row0, resrow):
            def body(r, accs):
                return tuple(
                    jnp.maximum(accs[c], gk[row0 + r, pl.ds(c * 32, 32)])
                    for c in range(BCH))

            init = tuple(gk[row0, pl.ds(c * 32, 32)] for c in range(BCH))
            accs = lax.fori_loop(1, LSEG, body, init)
            for c in range(BCH):
                res_v[resrow, pl.ds(c * 32, 32)] = accs[c]

        def block(i, carry):
            for j in range(NBUF):
                lq = i * NBUF + j
                pltpu.make_async_copy(
                    emb_hbm.at[idx_v.at[pl.ds(lq * QIDX, QIDX)]], gbufs[j],
                    sems[j]).wait()
                for j4 in range(QSEG):
                    reduce_seg(gbufs[j], j4 * LSEG, j * QSEG + j4)
                nxt = lq + NBUF

                @pl.when(nxt < q_w)
                def _():
                    fire(nxt, j)

            pltpu.sync_copy(
                res_v, out_hbm.at[pl.ds(seg_base + i * FLUSH, FLUSH)])
            return carry

        lax.fori_loop(0, nblk, block, 0)

    return pool(idx_flat, emb16)


def _tc_tail(rm, cm, w0m, w1m, wa, wb, bias, labels):
    bsz = rm.shape[0]

    def body(rm_ref, cm_ref, w0_ref, w1_ref, wa_ref, wb_ref, b_ref, lab_ref,
             loss_ref, logits_ref):
        def r16(x):
            return x.astype(jnp.bfloat16).astype(jnp.float32)

        args = r16(jnp.maximum(rm_ref[...].astype(jnp.float32),
                               cm_ref[...].astype(jnp.float32)))
        wav = r16(wa_ref[...])
        wbv = r16(wb_ref[...])
        aw = jnp.sum(args * wav, axis=1, keepdims=True)
        d0 = jnp.sum(r16(w0_ref[...].astype(jnp.float32)) * wbv, axis=1,
                     keepdims=True)
        d1 = jnp.sum(r16(w1_ref[...].astype(jnp.float32)) * wbv, axis=1,
                     keepdims=True)
        bb = b_ref[0, 0]
        l0 = aw + d0 + bb
        l1 = aw + d1 + bb
        m = jnp.maximum(l0, l1)
        lse = m + jnp.log(jnp.exp(l0 - m) + jnp.exp(l1 - m))
        logits_ref[...] = jnp.concatenate([l0, l1], axis=1)
        chosen = jnp.where(lab_ref[...] == 0, l0, l1)
        loss_ref[...] = jnp.mean(lse - chosen).reshape(1, 1)

    return pl.pallas_call(
        body,
        out_shape=(jax.ShapeDtypeStruct((1, 1), jnp.float32),
                   jax.ShapeDtypeStruct((bsz, 2), jnp.float32)),
    )(rm, cm, w0m, w1m, wa, wb, bias, labels)


def kernel(reasons, claims, warrant0s, warrant1s, label_ids, embeddings, W, b):
    bsz, lseq = reasons.shape
    idx_flat = jnp.concatenate(
        [reasons, claims, warrant0s, warrant1s], axis=0).reshape(-1)
    emb16 = jnp.pad(embeddings.astype(jnp.bfloat16), ((0, 0), (0, DP - D)))
    pooled = _sc_pool(idx_flat, emb16)

    rm = pooled[0 * bsz:1 * bsz, :D]
    cm = pooled[1 * bsz:2 * bsz, :D]
    w0m = pooled[2 * bsz:3 * bsz, :D]
    w1m = pooled[3 * bsz:4 * bsz, :D]
    wa = W[:D, 0].reshape(1, D)
    wb = W[D:, 0].reshape(1, D)
    bias = b.reshape(1, 1).astype(jnp.float32)
    labels = label_ids.reshape(bsz, 1)

    loss2d, logits = _tc_tail(rm, cm, w0m, w1m, wa, wb, bias, labels)
    return (loss2d[0, 0], logits)


# speedup vs baseline: 1.0014x; 1.0014x over previous
import functools

import jax
import jax.numpy as jnp
from jax import lax
from jax.experimental import pallas as pl
from jax.experimental.pallas import tpu as pltpu
from jax.experimental.pallas import tpu_sc as plsc

NC, NS = 2, 16
NW = NC * NS

D = 300
DP = 320
BCH = 10
LSEG = 50
QSEG = 4
QIDX = QSEG * LSEG
NBUF = 2
FLUSH = 8


def _sc_pool(idx_flat, emb16):
    nseg = idx_flat.shape[0] // LSEG
    seg_w = nseg // NW
    q_w = seg_w // QSEG
    nblk = q_w // NBUF

    mesh = plsc.VectorSubcoreMesh(core_axis_name="c", subcore_axis_name="s")

    @functools.partial(
        pl.kernel,
        mesh=mesh,
        compiler_params=pltpu.CompilerParams(use_tc_tiling_on_sc=False),
        out_type=jax.ShapeDtypeStruct((nseg, DP), jnp.bfloat16),
        scratch_types=[
            pltpu.VMEM((seg_w * LSEG,), jnp.int32),
            pltpu.VMEM((QIDX, DP), jnp.bfloat16),
            pltpu.VMEM((QIDX, DP), jnp.bfloat16),
            pltpu.VMEM((FLUSH, DP), jnp.bfloat16),
            pltpu.SemaphoreType.DMA,
            pltpu.SemaphoreType.DMA,
        ],
    )
    def pool(idx_hbm, emb_hbm, out_hbm, idx_v, g0, g1, res_v, s0, s1):
        gbufs = (g0, g1)
        sems = (s0, s1)
        wid = lax.axis_index("s") * NC + lax.axis_index("c")
        seg_base = wid * seg_w
        pltpu.sync_copy(idx_hbm.at[pl.ds(seg_base * LSEG, seg_w * LSEG)],
                        idx_v)

        def fire(lq, j):
            pltpu.async_copy(
                emb_hbm.at[idx_v.at[pl.ds(lq * QIDX, QIDX)]], gbufs[j],
                sems[j])

        for j in range(NBUF):
            fire(j, j)

        def reduce_seg(gk, row0, resrow):
            def body(r, accs):
                return tuple(
                    jnp.maximum(accs[c], gk[row0 + r, pl.ds(c * 32, 32)])
                    for c in range(BCH))

            init = tuple(gk[row0, pl.ds(c * 32, 32)] for c in range(BCH))
            accs = lax.fori_loop(1, LSEG, body, init)
            for c in range(BCH):
                res_v[resrow, pl.ds(c * 32, 32)] = accs[c]

        def block(i, carry):
            for j in range(NBUF):
                lq = i * NBUF + j
                pltpu.make_async_copy(
                    emb_hbm.at[idx_v.at[pl.ds(lq * QIDX, QIDX)]], gbufs[j],
                    sems[j]).wait()
                for j4 in range(QSEG):
                    reduce_seg(gbufs[j], j4 * LSEG, j * QSEG + j4)
                nxt = lq + NBUF

                @pl.when(nxt < q_w)
                def _():
                    fire(nxt, j)

            pltpu.sync_copy(
                res_v, out_hbm.at[pl.ds(seg_base + i * FLUSH, FLUSH)])
            return carry

        lax.fori_loop(0, nblk, block, 0)

    return pool(idx_flat, emb16)


def _tc_tail(rm, cm, w0m, w1m, wa, wb, bias, labels):
    bsz = rm.shape[0]

    def body(rm_ref, cm_ref, w0_ref, w1_ref, wa_ref, wb_ref, b_ref, lab_ref,
             loss_ref, logits_ref):
        def r16(x):
            return x.astype(jnp.bfloat16).astype(jnp.float32)

        args = r16(jnp.maximum(rm_ref[...].astype(jnp.float32),
                               cm_ref[...].astype(jnp.float32)))
        wav = r16(wa_ref[...])
        wbv = r16(wb_ref[...])
        aw = jnp.sum(args * wav, axis=1, keepdims=True)
        d0 = jnp.sum(r16(w0_ref[...].astype(jnp.float32)) * wbv, axis=1,
                     keepdims=True)
        d1 = jnp.sum(r16(w1_ref[...].astype(jnp.float32)) * wbv, axis=1,
                     keepdims=True)
        bb = b_ref[0, 0]
        l0 = aw + d0 + bb
        l1 = aw + d1 + bb
        m = jnp.maximum(l0, l1)
        lse = m + jnp.log(jnp.exp(l0 - m) + jnp.exp(l1 - m))
        logits_ref[...] = jnp.concatenate([l0, l1], axis=1)
        chosen = jnp.where(lab_ref[...] == 0, l0, l1)
        loss_ref[...] = jnp.mean(lse - chosen).reshape(1, 1)

    return pl.pallas_call(
        body,
        out_shape=(jax.ShapeDtypeStruct((1, 1), jnp.float32),
                   jax.ShapeDtypeStruct((bsz, 2), jnp.float32)),
    )(rm, cm, w0m, w1m, wa, wb, bias, labels)


def kernel(reasons, claims, warrant0s, warrant1s, label_ids, embeddings, W, b):
    bsz, lseq = reasons.shape
    idx_flat = jnp.concatenate(
        [reasons, claims, warrant0s, warrant1s], axis=0).reshape(-1)
    emb16 = jnp.pad(embeddings, ((0, 0), (0, DP - D))).astype(jnp.bfloat16)
    pooled = _sc_pool(idx_flat, emb16)

    rm = pooled[0 * bsz:1 * bsz, :D]
    cm = pooled[1 * bsz:2 * bsz, :D]
    w0m = pooled[2 * bsz:3 * bsz, :D]
    w1m = pooled[3 * bsz:4 * bsz, :D]
    wa = W[:D, 0].reshape(1, D)
    wb = W[D:, 0].reshape(1, D)
    bias = b.reshape(1, 1).astype(jnp.float32)
    labels = label_ids.reshape(bsz, 1)

    loss2d, logits = _tc_tail(rm, cm, w0m, w1m, wa, wb, bias, labels)
    return (loss2d[0, 0], logits)
